# speedup vs baseline: 1.4271x; 1.4271x over previous
import functools

import jax
import jax.numpy as jnp
from jax import lax
from jax.experimental import pallas as pl
from jax.experimental.pallas import tpu as pltpu
from jax.experimental.pallas import tpu_sc as plsc

D_VOCAB = 50257
D_MODEL = 768
BATCH = 4
SEQ = 2048

NC = 2
NS = 16
NW = NC * NS

B_TOTAL = BATCH * SEQ
B_PER_W = B_TOTAL // NW
CHUNK = 64
NCHUNK = B_PER_W // CHUNK


def _embed_body(table_hbm, tok_hbm, out_hbm, idx_v, rows_v, gsem, osem):
    wid = lax.axis_index("s") * NC + lax.axis_index("c")
    pltpu.sync_copy(tok_hbm.at[wid], idx_v)
    out_copies = [None] * NCHUNK
    for c in range(NCHUNK):
        b = c % 2
        if c >= 2:
            out_copies[c - 2].wait()
        pltpu.async_copy(table_hbm.at[idx_v.at[c]], rows_v.at[b], gsem).wait()
        base = (wid * NCHUNK + c) * CHUNK
        out_copies[c] = pltpu.async_copy(
            rows_v.at[b], out_hbm.at[pl.ds(base, CHUNK)], osem
        )
    for c in range(NCHUNK - 2, NCHUNK):
        out_copies[c].wait()


@jax.jit
def _embed(tokens_grouped, W_E):
    mesh = plsc.VectorSubcoreMesh(core_axis_name="c", subcore_axis_name="s")
    run = functools.partial(
        pl.kernel,
        mesh=mesh,
        out_type=jax.ShapeDtypeStruct((B_TOTAL, D_MODEL), jnp.float32),
        scratch_types=[
            pltpu.VMEM((NCHUNK, CHUNK), jnp.int32),
            pltpu.VMEM((2, CHUNK, D_MODEL), jnp.float32),
            pltpu.SemaphoreType.DMA,
            pltpu.SemaphoreType.DMA,
        ],
    )(_embed_body)
    return run(W_E, tokens_grouped)


def kernel(tokens, W_E):
    tok = tokens.astype(jnp.int32).reshape(NW, NCHUNK, CHUNK)
    out = _embed(tok, W_E)
    return out.reshape(BATCH, SEQ, D_MODEL)


# speedup vs baseline: 1.5102x; 1.0583x over previous
import functools

import jax
import jax.numpy as jnp
from jax import lax
from jax.experimental import pallas as pl
from jax.experimental.pallas import tpu as pltpu
from jax.experimental.pallas import tpu_sc as plsc

D_VOCAB = 50257
D_MODEL = 768
BATCH = 4
SEQ = 2048

NC = 2
NS = 16
NW = NC * NS

B_TOTAL = BATCH * SEQ
B_PER_W = B_TOTAL // NW
CHUNK = 32
NCHUNK = B_PER_W // CHUNK
NBUF = 4
DEPTH = 2


def _embed_body(table_hbm, tok_hbm, out_hbm, idx_v, rows_v, gsems, osems):
    wid = lax.axis_index("s") * NC + lax.axis_index("c")
    pltpu.sync_copy(tok_hbm.at[wid], idx_v)

    gathers = [None] * NCHUNK
    out_copies = [None] * NCHUNK
    out_waited = [False] * NCHUNK

    def start_gather(c):
        b = c % NBUF
        gathers[c] = pltpu.async_copy(
            table_hbm.at[idx_v.at[c]], rows_v.at[b], gsems.at[b]
        )

    for c in range(DEPTH):
        start_gather(c)
    for c in range(NCHUNK):
        b = c % NBUF
        nc = c + DEPTH
        if nc < NCHUNK:
            prev = nc - NBUF
            if prev >= 0:
                out_copies[prev].wait()
                out_waited[prev] = True
            start_gather(nc)
        gathers[c].wait()
        base = (wid * NCHUNK + c) * CHUNK
        out_copies[c] = pltpu.async_copy(
            rows_v.at[b], out_hbm.at[pl.ds(base, CHUNK)], osems.at[b]
        )
    for c in range(NCHUNK):
        if not out_waited[c]:
            out_copies[c].wait()


@jax.jit
def _embed(tokens_grouped, W_E):
    mesh = plsc.VectorSubcoreMesh(core_axis_name="c", subcore_axis_name="s")
    run = functools.partial(
        pl.kernel,
        mesh=mesh,
        out_type=jax.ShapeDtypeStruct((B_TOTAL, D_MODEL), jnp.float32),
        scratch_types=[
            pltpu.VMEM((NCHUNK, CHUNK), jnp.int32),
            pltpu.VMEM((NBUF, CHUNK, D_MODEL), jnp.float32),
            pltpu.SemaphoreType.DMA((NBUF,)),
            pltpu.SemaphoreType.DMA((NBUF,)),
        ],
    )(_embed_body)
    return run(W_E, tokens_grouped)


def kernel(tokens, W_E):
    tok = tokens.astype(jnp.int32).reshape(NW, NCHUNK, CHUNK)
    out = _embed(tok, W_E)
    return out.reshape(BATCH, SEQ, D_MODEL)


# speedup vs baseline: 1.5184x; 1.0054x over previous
import functools

import jax
import jax.numpy as jnp
from jax import lax
from jax.experimental import pallas as pl
from jax.experimental.pallas import tpu as pltpu
from jax.experimental.pallas import tpu_sc as plsc

D_VOCAB = 50257
D_MODEL = 768
BATCH = 4
SEQ = 2048

NC = 2
NS = 16
NW = NC * NS

B_TOTAL = BATCH * SEQ
B_PER_W = B_TOTAL // NW
WPB = SEQ // B_PER_W
CHUNK = 32
NCHUNK = B_PER_W // CHUNK
NBUF = 4
DEPTH = 2


def _embed_body(table_hbm, tok_hbm, out_hbm, idx_v, rows_v, gsems, osems):
    wid = lax.axis_index("s") * NC + lax.axis_index("c")
    bidx = wid // WPB
    boff = (wid % WPB) * B_PER_W
    pltpu.sync_copy(tok_hbm.at[bidx, pl.ds(boff, B_PER_W)], idx_v)

    gathers = [None] * NCHUNK
    out_copies = [None] * NCHUNK
    out_waited = [False] * NCHUNK

    def start_gather(c):
        b = c % NBUF
        gathers[c] = pltpu.async_copy(
            table_hbm.at[idx_v.at[pl.ds(c * CHUNK, CHUNK)]],
            rows_v.at[b],
            gsems.at[b],
        )

    for c in range(DEPTH):
        start_gather(c)
    for c in range(NCHUNK):
        b = c % NBUF
        nc = c + DEPTH
        if nc < NCHUNK:
            prev = nc - NBUF
            if prev >= 0:
                out_copies[prev].wait()
                out_waited[prev] = True
            start_gather(nc)
        gathers[c].wait()
        out_copies[c] = pltpu.async_copy(
            rows_v.at[b],
            out_hbm.at[bidx, pl.ds(boff + c * CHUNK, CHUNK)],
            osems.at[b],
        )
    for c in range(NCHUNK):
        if not out_waited[c]:
            out_copies[c].wait()


@jax.jit
def _embed(tokens, W_E):
    mesh = plsc.VectorSubcoreMesh(core_axis_name="c", subcore_axis_name="s")
    run = functools.partial(
        pl.kernel,
        mesh=mesh,
        out_type=jax.ShapeDtypeStruct((BATCH, SEQ, D_MODEL), jnp.float32),
        scratch_types=[
            pltpu.VMEM((B_PER_W,), jnp.int32),
            pltpu.VMEM((NBUF, CHUNK, D_MODEL), jnp.float32),
            pltpu.SemaphoreType.DMA((NBUF,)),
            pltpu.SemaphoreType.DMA((NBUF,)),
        ],
    )(_embed_body)
    return run(W_E, tokens)


def kernel(tokens, W_E):
    return _embed(tokens.astype(jnp.int32), W_E)
